# Initial kernel scaffold; baseline (speedup 1.0000x reference)
import jax
import jax.numpy as jnp
from jax.experimental import pallas as pl


def kernel(X, edge_index, W1, b1, W2, b2, Wp1, bp1, Wp2, bp2):
    raise NotImplementedError("write your pallas kernel here")


# speedup vs baseline: 10.7686x; 10.7686x over previous
import functools

import jax
import jax.numpy as jnp
from jax import lax
from jax.experimental import pallas as pl
from jax.experimental.pallas import tpu as pltpu
from jax.experimental.pallas import tpu_sc as plsc

N = 10000
D = 128
E = 320000
NSC = 2
NSUB = 16
NT = NSC * NSUB
CH = 128
EPT = 10112
EP = NT * EPT
NCH = EPT // CH
NROW = 10240
RPT = NROW // NSUB
ZR = 64


def _deg_body(dstp, out, dst_v, ones_v, zb, deg_sh):
  c = lax.axis_index("c")
  s = lax.axis_index("s")
  w = c * NSUB + s

  def fill_ones(i, _):
    ones_v[i, :] = jnp.ones((16,), jnp.float32)
    return _
  lax.fori_loop(0, CH, fill_ones, None)

  def fill_zero(i, _):
    zb[i, :] = jnp.zeros((16,), jnp.float32)
    return _
  lax.fori_loop(0, ZR, fill_zero, None)

  def zero_acc(i, _):
    pltpu.sync_copy(zb, deg_sh.at[pl.ds(s * RPT + i * ZR, ZR)])
    return _
  lax.fori_loop(0, RPT // ZR, zero_acc, None)
  plsc.subcore_barrier()

  base0 = w * EPT

  def step(i, _):
    base = pl.multiple_of(base0 + i * CH, 8)
    pltpu.sync_copy(dstp.at[pl.ds(base, CH)], dst_v)
    pltpu.sync_copy(ones_v, deg_sh.at[dst_v], add=True)
    return _
  lax.fori_loop(0, NCH, step, None)
  plsc.subcore_barrier()

  pltpu.sync_copy(deg_sh.at[pl.ds(s * RPT, RPT)], out.at[c, pl.ds(s * RPT, RPT)])


_deg_call = pl.kernel(
    _deg_body,
    out_type=jax.ShapeDtypeStruct((NSC, NROW, 16), jnp.float32),
    mesh=plsc.VectorSubcoreMesh(core_axis_name="c", subcore_axis_name="s"),
    scratch_types=[
        pltpu.VMEM((CH,), jnp.int32),
        pltpu.VMEM((CH, 16), jnp.float32),
        pltpu.VMEM((ZR, 16), jnp.float32),
        pltpu.VMEM_SHARED((NROW, 16), jnp.float32),
    ],
)


def _agg_body(g, srcp, dstp, out, src_v, dst_v, rows_v, zb, acc_sh, sem):
  c = lax.axis_index("c")
  s = lax.axis_index("s")
  w = c * NSUB + s

  def fill_zero(i, _):
    r = i // 8
    j = i % 8
    zb[r, pl.ds(j * 16, 16)] = jnp.zeros((16,), jnp.float32)
    return _
  lax.fori_loop(0, ZR * 8, fill_zero, None)

  def zero_acc(i, _):
    pltpu.sync_copy(zb, acc_sh.at[pl.ds(s * RPT + i * ZR, ZR)])
    return _
  lax.fori_loop(0, RPT // ZR, zero_acc, None)
  plsc.subcore_barrier()

  base0 = w * EPT

  def step(i, _):
    base = pl.multiple_of(base0 + i * CH, 8)
    pltpu.sync_copy(srcp.at[pl.ds(base, CH)], src_v)
    pltpu.sync_copy(dstp.at[pl.ds(base, CH)], dst_v)
    pltpu.async_copy(g.at[src_v], rows_v, sem).wait()
    pltpu.sync_copy(rows_v, acc_sh.at[dst_v], add=True)
    return _
  lax.fori_loop(0, NCH, step, None)
  plsc.subcore_barrier()

  pltpu.sync_copy(acc_sh.at[pl.ds(s * RPT, RPT)], out.at[c, pl.ds(s * RPT, RPT)])


_agg_call = pl.kernel(
    _agg_body,
    out_type=jax.ShapeDtypeStruct((NSC, NROW, D), jnp.float32),
    mesh=plsc.VectorSubcoreMesh(core_axis_name="c", subcore_axis_name="s"),
    scratch_types=[
        pltpu.VMEM((CH,), jnp.int32),
        pltpu.VMEM((CH,), jnp.int32),
        pltpu.VMEM((CH, D), jnp.float32),
        pltpu.VMEM((ZR, D), jnp.float32),
        pltpu.VMEM_SHARED((NROW, D), jnp.float32),
        pltpu.SemaphoreType.DMA,
    ],
)

BM = 1000


def _dinv(degc):
  deg = (degc[0] + degc[1])[:, 0:1] + 1.0
  return lax.rsqrt(deg)


def _tc1_body(degc, x, w1, g1):
  dinv = _dinv(degc)
  g1[...] = jnp.dot(x[...], w1[...], preferred_element_type=jnp.float32) * dinv


def _tc2_body(degc, agg1, g1, b1, w2, g2):
  dinv = _dinv(degc)
  a = agg1[0] + agg1[1] + g1[...]
  x2 = jnp.maximum(a * dinv + b1[...], 0.0)
  g2[...] = jnp.dot(x2, w2[...], preferred_element_type=jnp.float32) * dinv


def _tc3_body(degc, agg2, g2, b2, wp1, bp1, wp2, bp2, z):
  dinv = _dinv(degc)
  h = (agg2[0] + agg2[1] + g2[...]) * dinv + b2[...]
  t = jnp.maximum(
      jnp.dot(h, wp1[...], preferred_element_type=jnp.float32) + bp1[...], 0.0)
  z[...] = jnp.dot(t, wp2[...], preferred_element_type=jnp.float32) + bp2[...]


def _degc_spec():
  return pl.BlockSpec((NSC, BM, 16), lambda i: (0, i, 0))


def _rows_spec(d=D):
  return pl.BlockSpec((BM, d), lambda i: (i, 0))


def _agg_spec():
  return pl.BlockSpec((NSC, BM, D), lambda i: (0, i, 0))


def _full_spec(r, c):
  return pl.BlockSpec((r, c), lambda i: (0, 0))


_tc1 = pl.pallas_call(
    _tc1_body,
    grid=(N // BM,),
    in_specs=[_degc_spec(), _rows_spec(), _full_spec(D, D)],
    out_specs=_rows_spec(),
    out_shape=jax.ShapeDtypeStruct((N, D), jnp.float32),
)

_tc2 = pl.pallas_call(
    _tc2_body,
    grid=(N // BM,),
    in_specs=[_degc_spec(), _agg_spec(), _rows_spec(), _full_spec(1, D),
              _full_spec(D, D)],
    out_specs=_rows_spec(),
    out_shape=jax.ShapeDtypeStruct((N, D), jnp.float32),
)

_tc3 = pl.pallas_call(
    _tc3_body,
    grid=(N // BM,),
    in_specs=[_degc_spec(), _agg_spec(), _rows_spec(), _full_spec(1, D),
              _full_spec(D, 64), _full_spec(1, 64), _full_spec(64, D),
              _full_spec(1, D)],
    out_specs=_rows_spec(),
    out_shape=jax.ShapeDtypeStruct((N, D), jnp.float32),
)


def kernel(X, edge_index, W1, b1, W2, b2, Wp1, bp1, Wp2, bp2):
  ei = edge_index.astype(jnp.int32)
  src = jnp.concatenate([ei[0], jnp.zeros((EP - E,), jnp.int32)])
  dst = jnp.concatenate([ei[1], jnp.full((EP - E,), N, jnp.int32)])

  degc = _deg_call(dst)[:, :N]
  g1 = _tc1(degc, X, W1)
  agg1 = _agg_call(g1, src, dst)[:, :N]
  g2 = _tc2(degc, agg1, g1, b1.reshape(1, D), W2)
  agg2 = _agg_call(g2, src, dst)[:, :N]
  z = _tc3(degc, agg2, g2, b2.reshape(1, D), Wp1, bp1.reshape(1, 64), Wp2,
           bp2.reshape(1, D))
  return z


# speedup vs baseline: 12.8132x; 1.1899x over previous
import jax
import jax.numpy as jnp
from jax import lax
from jax.experimental import pallas as pl
from jax.experimental.pallas import tpu as pltpu
from jax.experimental.pallas import tpu_sc as plsc

N = 10000
D = 128
E = 320000
NSC = 2
NSUB = 16
NT = NSC * NSUB
CH = 128
EPT = 10112
EP = NT * EPT
NCH = EPT // CH
NROW = 10240
RPT = NROW // NSUB
ZR = 64


def _deg_body(dstp, out, dst_v, ones_v, zb, deg_sh):
  c = lax.axis_index("c")
  s = lax.axis_index("s")
  w = c * NSUB + s

  def fill_ones(i, _):
    ones_v[i, :] = jnp.ones((16,), jnp.float32)
    return _
  lax.fori_loop(0, CH, fill_ones, None)

  def fill_zero(i, _):
    zb[i, :] = jnp.zeros((16,), jnp.float32)
    return _
  lax.fori_loop(0, ZR, fill_zero, None)

  def zero_acc(i, _):
    pltpu.sync_copy(zb, deg_sh.at[pl.ds(s * RPT + i * ZR, ZR)])
    return _
  lax.fori_loop(0, RPT // ZR, zero_acc, None)
  plsc.subcore_barrier()

  base0 = w * EPT

  def step(i, _):
    base = pl.multiple_of(base0 + i * CH, 8)
    pltpu.sync_copy(dstp.at[pl.ds(base, CH)], dst_v)
    pltpu.sync_copy(ones_v, deg_sh.at[dst_v], add=True)
    return _
  lax.fori_loop(0, NCH, step, None)
  plsc.subcore_barrier()

  pltpu.sync_copy(deg_sh.at[pl.ds(s * RPT, RPT)], out.at[c, pl.ds(s * RPT, RPT)])


_deg_call = pl.kernel(
    _deg_body,
    out_type=jax.ShapeDtypeStruct((NSC, NROW, 16), jnp.float32),
    mesh=plsc.VectorSubcoreMesh(core_axis_name="c", subcore_axis_name="s"),
    scratch_types=[
        pltpu.VMEM((CH,), jnp.int32),
        pltpu.VMEM((CH, 16), jnp.float32),
        pltpu.VMEM((ZR, 16), jnp.float32),
        pltpu.VMEM_SHARED((NROW, 16), jnp.float32),
    ],
)


def _agg_body(g, srcp, dstp, out, sv0, sv1, dv0, dv1, rv0, rv1, zb, acc_sh,
              gs0, gs1):
  c = lax.axis_index("c")
  s = lax.axis_index("s")
  w = c * NSUB + s
  src_vs = [sv0, sv1]
  dst_vs = [dv0, dv1]
  rows = [rv0, rv1]
  gsems = [gs0, gs1]

  def fill_zero(i, _):
    r = i // 8
    j = i % 8
    zb[r, pl.ds(j * 16, 16)] = jnp.zeros((16,), jnp.float32)
    return _
  lax.fori_loop(0, ZR * 8, fill_zero, None)

  def zero_acc(i, _):
    pltpu.sync_copy(zb, acc_sh.at[pl.ds(s * RPT + i * ZR, ZR)])
    return _
  lax.fori_loop(0, RPT // ZR, zero_acc, None)
  plsc.subcore_barrier()

  base0 = w * EPT

  def epoch(o, _):
    gw = []
    for b in range(2):
      base = pl.multiple_of(base0 + (o * 2 + b) * CH, 8)
      pltpu.sync_copy(srcp.at[pl.ds(base, CH)], src_vs[b])
      pltpu.sync_copy(dstp.at[pl.ds(base, CH)], dst_vs[b])
      gw.append(pltpu.async_copy(g.at[src_vs[b]], rows[b], gsems[b]))
    for b in range(2):
      gw[b].wait()
      pltpu.sync_copy(rows[b], acc_sh.at[dst_vs[b]], add=True)
    return _
  lax.fori_loop(0, NCH // 2, epoch, None)
  plsc.subcore_barrier()

  pltpu.sync_copy(acc_sh.at[pl.ds(s * RPT, RPT)], out.at[c, pl.ds(s * RPT, RPT)])


_agg_call = pl.kernel(
    _agg_body,
    out_type=jax.ShapeDtypeStruct((NSC, NROW, D), jnp.float32),
    mesh=plsc.VectorSubcoreMesh(core_axis_name="c", subcore_axis_name="s"),
    scratch_types=(
        [pltpu.VMEM((CH,), jnp.int32)] * 4 +
        [pltpu.VMEM((CH, D), jnp.float32)] * 2 +
        [pltpu.VMEM((ZR, D), jnp.float32),
         pltpu.VMEM_SHARED((NROW, D), jnp.float32)] +
        [pltpu.SemaphoreType.DMA] * 2),
)

BM = 1000


def _dinv(degc):
  deg = (degc[0] + degc[1])[:, 0:1] + 1.0
  return lax.rsqrt(deg)


def _tc1_body(degc, x, w1, g1):
  dinv = _dinv(degc)
  g1[...] = jnp.dot(x[...], w1[...], preferred_element_type=jnp.float32) * dinv


def _tc2_body(degc, agg1, g1, b1, w2, g2):
  dinv = _dinv(degc)
  a = agg1[0] + agg1[1] + g1[...]
  x2 = jnp.maximum(a * dinv + b1[...], 0.0)
  g2[...] = jnp.dot(x2, w2[...], preferred_element_type=jnp.float32) * dinv


def _tc3_body(degc, agg2, g2, b2, wp1, bp1, wp2, bp2, z):
  dinv = _dinv(degc)
  a = agg2[0] + agg2[1] + g2[...]
  h = a * dinv + b2[...]
  t = jnp.maximum(
      jnp.dot(h, wp1[...], preferred_element_type=jnp.float32) + bp1[...], 0.0)
  z[...] = jnp.dot(t, wp2[...], preferred_element_type=jnp.float32) + bp2[...]


def _degc_spec():
  return pl.BlockSpec((NSC, BM, 16), lambda i: (0, i, 0))


def _rows_spec(d=D):
  return pl.BlockSpec((BM, d), lambda i: (i, 0))


def _agg_spec():
  return pl.BlockSpec((NSC, BM, D), lambda i: (0, i, 0))


def _full_spec(r, c):
  return pl.BlockSpec((r, c), lambda i: (0, 0))


_tc1 = pl.pallas_call(
    _tc1_body,
    grid=(N // BM,),
    in_specs=[_degc_spec(), _rows_spec(), _full_spec(D, D)],
    out_specs=_rows_spec(),
    out_shape=jax.ShapeDtypeStruct((N, D), jnp.float32),
)

_tc2 = pl.pallas_call(
    _tc2_body,
    grid=(N // BM,),
    in_specs=[_degc_spec(), _agg_spec(), _rows_spec(), _full_spec(1, D),
              _full_spec(D, D)],
    out_specs=_rows_spec(),
    out_shape=jax.ShapeDtypeStruct((N, D), jnp.float32),
)

_tc3 = pl.pallas_call(
    _tc3_body,
    grid=(N // BM,),
    in_specs=[_degc_spec(), _agg_spec(), _rows_spec(), _full_spec(1, D),
              _full_spec(D, 64), _full_spec(1, 64), _full_spec(64, D),
              _full_spec(1, D)],
    out_specs=_rows_spec(),
    out_shape=jax.ShapeDtypeStruct((N, D), jnp.float32),
)


def kernel(X, edge_index, W1, b1, W2, b2, Wp1, bp1, Wp2, bp2):
  ei = edge_index.astype(jnp.int32)
  src = jnp.concatenate([ei[0], jnp.zeros((EP - E,), jnp.int32)])
  dst = jnp.concatenate([ei[1], jnp.full((EP - E,), N, jnp.int32)])

  degc = _deg_call(dst)[:, :N]
  g1 = _tc1(degc, X, W1)
  agg1 = _agg_call(g1, src, dst)[:, :N]
  g2 = _tc2(degc, agg1, g1, b1.reshape(1, D), W2)
  agg2 = _agg_call(g2, src, dst)[:, :N]
  z = _tc3(degc, agg2, g2, b2.reshape(1, D), Wp1, bp1.reshape(1, 64), Wp2,
           bp2.reshape(1, D))
  return z


# speedup vs baseline: 12.8228x; 1.0007x over previous
import jax
import jax.numpy as jnp
from jax import lax
from jax.experimental import pallas as pl
from jax.experimental.pallas import tpu as pltpu
from jax.experimental.pallas import tpu_sc as plsc

N = 10000
D = 128
E = 320000
NSC = 2
NSUB = 16
NT = NSC * NSUB
CH = 128
EPT = 10112
EP = NT * EPT
NCH = EPT // CH
NROW = 10240
RPT = NROW // NSUB
ZR = 64


def _deg_body(dstp, out, dst_v, ones_v, zb, deg_sh):
  c = lax.axis_index("c")
  s = lax.axis_index("s")
  w = c * NSUB + s

  def fill_ones(i, _):
    ones_v[i, :] = jnp.ones((16,), jnp.float32)
    return _
  lax.fori_loop(0, CH, fill_ones, None)

  def fill_zero(i, _):
    zb[i, :] = jnp.zeros((16,), jnp.float32)
    return _
  lax.fori_loop(0, ZR, fill_zero, None)

  def zero_acc(i, _):
    pltpu.sync_copy(zb, deg_sh.at[pl.ds(s * RPT + i * ZR, ZR)])
    return _
  lax.fori_loop(0, RPT // ZR, zero_acc, None)
  plsc.subcore_barrier()

  base0 = w * EPT

  def step(i, _):
    base = pl.multiple_of(base0 + i * CH, 8)
    pltpu.sync_copy(dstp.at[pl.ds(base, CH)], dst_v)
    pltpu.sync_copy(ones_v, deg_sh.at[dst_v], add=True)
    return _
  lax.fori_loop(0, NCH, step, None)
  plsc.subcore_barrier()

  pltpu.sync_copy(deg_sh.at[pl.ds(s * RPT, RPT)], out.at[c, pl.ds(s * RPT, RPT)])


_deg_call = pl.kernel(
    _deg_body,
    out_type=jax.ShapeDtypeStruct((NSC, NROW, 16), jnp.float32),
    mesh=plsc.VectorSubcoreMesh(core_axis_name="c", subcore_axis_name="s"),
    scratch_types=[
        pltpu.VMEM((CH,), jnp.int32),
        pltpu.VMEM((CH, 16), jnp.float32),
        pltpu.VMEM((ZR, 16), jnp.float32),
        pltpu.VMEM_SHARED((NROW, 16), jnp.float32),
    ],
)


def _agg_body(g, srcp, dstp, out, sv0, sv1, dv0, dv1, rv0, rv1, zb, acc_sh,
              gs0, gs1):
  c = lax.axis_index("c")
  s = lax.axis_index("s")
  w = c * NSUB + s
  src_vs = [sv0, sv1]
  dst_vs = [dv0, dv1]
  rows = [rv0, rv1]
  gsems = [gs0, gs1]

  def fill_zero(i, _):
    r = i // 8
    j = i % 8
    zb[r, pl.ds(j * 16, 16)] = jnp.zeros((16,), jnp.float32)
    return _
  lax.fori_loop(0, ZR * 8, fill_zero, None)

  def zero_acc(i, _):
    pltpu.sync_copy(zb, acc_sh.at[pl.ds(s * RPT + i * ZR, ZR)])
    return _
  lax.fori_loop(0, RPT // ZR, zero_acc, None)
  plsc.subcore_barrier()

  base0 = w * EPT

  def epoch(o, _):
    gw = []
    for b in range(2):
      base = pl.multiple_of(base0 + (o * 2 + b) * CH, 8)
      pltpu.sync_copy(srcp.at[pl.ds(base, CH)], src_vs[b])
      pltpu.sync_copy(dstp.at[pl.ds(base, CH)], dst_vs[b])
      gw.append(pltpu.async_copy(g.at[src_vs[b]], rows[b], gsems[b]))
    for b in range(2):
      gw[b].wait()
      pltpu.sync_copy(rows[b], acc_sh.at[dst_vs[b]], add=True)
    return _
  lax.fori_loop(0, NCH // 2, epoch, None)
  plsc.subcore_barrier()

  pltpu.sync_copy(acc_sh.at[pl.ds(s * RPT, RPT)], out.at[c, pl.ds(s * RPT, RPT)])


_agg_call = pl.kernel(
    _agg_body,
    out_type=jax.ShapeDtypeStruct((NSC, NROW, D), jnp.float32),
    mesh=plsc.VectorSubcoreMesh(core_axis_name="c", subcore_axis_name="s"),
    scratch_types=(
        [pltpu.VMEM((CH,), jnp.int32) for _ in range(4)] +
        [pltpu.VMEM((CH, D), jnp.float32) for _ in range(2)] +
        [pltpu.VMEM((ZR, D), jnp.float32),
         pltpu.VMEM_SHARED((NROW, D), jnp.float32)] +
        [pltpu.SemaphoreType.DMA for _ in range(2)]),
)

BM = 1000


def _dinv(degc):
  deg = (degc[0] + degc[1])[:, 0:1] + 1.0
  return lax.rsqrt(deg)


def _tc1_body(degc, x, w1, g1):
  dinv = _dinv(degc)
  g1[...] = jnp.dot(x[...], w1[...], preferred_element_type=jnp.float32) * dinv


def _tc2_body(degc, agg1, g1, b1, w2, g2):
  dinv = _dinv(degc)
  a = agg1[0] + agg1[1] + g1[...]
  x2 = jnp.maximum(a * dinv + b1[...], 0.0)
  g2[...] = jnp.dot(x2, w2[...], preferred_element_type=jnp.float32) * dinv


def _tc3_body(degc, agg2, g2, b2, wp1, bp1, wp2, bp2, z):
  dinv = _dinv(degc)
  a = agg2[0] + agg2[1] + g2[...]
  h = a * dinv + b2[...]
  t = jnp.maximum(
      jnp.dot(h, wp1[...], preferred_element_type=jnp.float32) + bp1[...], 0.0)
  z[...] = jnp.dot(t, wp2[...], preferred_element_type=jnp.float32) + bp2[...]


def _degc_spec():
  return pl.BlockSpec((NSC, BM, 16), lambda i: (0, i, 0))


def _rows_spec(d=D):
  return pl.BlockSpec((BM, d), lambda i: (i, 0))


def _agg_spec():
  return pl.BlockSpec((NSC, BM, D), lambda i: (0, i, 0))


def _full_spec(r, c):
  return pl.BlockSpec((r, c), lambda i: (0, 0))


_tc1 = pl.pallas_call(
    _tc1_body,
    grid=(N // BM,),
    in_specs=[_degc_spec(), _rows_spec(), _full_spec(D, D)],
    out_specs=_rows_spec(),
    out_shape=jax.ShapeDtypeStruct((N, D), jnp.float32),
)

_tc2 = pl.pallas_call(
    _tc2_body,
    grid=(N // BM,),
    in_specs=[_degc_spec(), _agg_spec(), _rows_spec(), _full_spec(1, D),
              _full_spec(D, D)],
    out_specs=_rows_spec(),
    out_shape=jax.ShapeDtypeStruct((N, D), jnp.float32),
)

_tc3 = pl.pallas_call(
    _tc3_body,
    grid=(N // BM,),
    in_specs=[_degc_spec(), _agg_spec(), _rows_spec(), _full_spec(1, D),
              _full_spec(D, 64), _full_spec(1, 64), _full_spec(64, D),
              _full_spec(1, D)],
    out_specs=_rows_spec(),
    out_shape=jax.ShapeDtypeStruct((N, D), jnp.float32),
)


def kernel(X, edge_index, W1, b1, W2, b2, Wp1, bp1, Wp2, bp2):
  ei = edge_index.astype(jnp.int32)
  src = jnp.concatenate([ei[0], jnp.zeros((EP - E,), jnp.int32)])
  dst = jnp.concatenate([ei[1], jnp.full((EP - E,), N, jnp.int32)])

  degc = _deg_call(dst)[:, :N]
  g1 = _tc1(degc, X, W1)
  agg1 = _agg_call(g1, src, dst)[:, :N]
  g2 = _tc2(degc, agg1, g1, b1.reshape(1, D), W2)
  agg2 = _agg_call(g2, src, dst)[:, :N]
  z = _tc3(degc, agg2, g2, b2.reshape(1, D), Wp1, bp1.reshape(1, 64), Wp2,
           bp2.reshape(1, D))
  return z
